# baseline (speedup 1.0000x reference)
import functools

import jax
import jax.numpy as jnp
from jax import lax
from jax.experimental import pallas as pl
from jax.experimental.pallas import tpu as pltpu
from jax.experimental.pallas import tpu_sc as plsc

ALPHA = 0.001
BETA = 0.001
S = 3
R_DIM = 64
DF_DIM = 448
DP = 512
MP = 128
L = 16


def _tc_pack_d(d_t, bs):
    F, N = d_t.shape
    grid = ((N + bs - 1) // bs,)

    def body(i_ref, o_ref):
        for g in range(4):
            w = min(128, F - g * 128)
            o_ref[:, g, 0:w] = i_ref[g * 128:g * 128 + w, :].T

    return pl.pallas_call(
        body,
        grid=grid,
        in_specs=[pl.BlockSpec((F, bs), lambda b: (0, b))],
        out_specs=pl.BlockSpec((bs, 4, 128), lambda b: (b, 0, 0)),
        out_shape=jax.ShapeDtypeStruct((N, 4, 128), d_t.dtype),
    )(d_t)


def _tc_pack_m(m_t, bs):
    F, N = m_t.shape
    grid = ((N + bs - 1) // bs,)

    def body(i_ref, o_ref):
        o_ref[:, 0:F] = i_ref[...].T

    return pl.pallas_call(
        body,
        grid=grid,
        in_specs=[pl.BlockSpec((F, bs), lambda b: (0, b))],
        out_specs=pl.BlockSpec((bs, MP), lambda b: (b, 0)),
        out_shape=jax.ShapeDtypeStruct((N, MP), m_t.dtype),
    )(m_t)


def kernel(ijk, m_bar, d_bar, M, D_full):
    B = ijk.shape[0]
    info = plsc.get_sparse_core_info()
    NC, NS = info.num_cores, info.num_subcores
    NW = NC * NS
    EPW = B // NW
    C = 64
    NCH = EPW // C

    mesh = plsc.VectorSubcoreMesh(core_axis_name="c", subcore_axis_name="s")

    @functools.partial(
        pl.kernel,
        mesh=mesh,
        out_type=jax.ShapeDtypeStruct((B,), jnp.float32),
        compiler_params=pltpu.CompilerParams(
            use_tc_tiling_on_sc=False, needs_layout_passes=False),
        scratch_types=[
            pltpu.VMEM((C,), jnp.int32),
            pltpu.VMEM((C,), jnp.int32),
            pltpu.VMEM((C,), jnp.int32),
            pltpu.VMEM((C,), jnp.int32),
            pltpu.VMEM((C,), jnp.int32),
            pltpu.VMEM((4 * C,), jnp.int32),
            pltpu.VMEM((C,), jnp.float32),
            pltpu.VMEM((C,), jnp.float32),
            pltpu.VMEM((C, MP), jnp.float32),
            pltpu.VMEM((C, MP), jnp.float32),
            pltpu.VMEM((C, MP), jnp.float32),
            pltpu.VMEM((C, MP), jnp.float32),
            pltpu.VMEM((4 * C, 128), jnp.float32),
            pltpu.VMEM((C,), jnp.float32),
            pltpu.SemaphoreType.DMA,
        ],
    )
    def sc_kernel(iv_hbm, jv_hbm, k0_hbm, k1_hbm, k2_hbm,
                  mbar_hbm, dbar_hbm, M_hbm, Df_hbm, out_hbm,
                  iv, jv, k0v, k1v, k2v, jv4, mb_v, db_v,
                  Mi_v, Mk0_v, Mk1_v, Mk2_v, Df_v, out_v, sem):
        wid = lax.axis_index("s") * NC + lax.axis_index("c")

        def chunk_body(ch, _):
            base = pl.multiple_of(wid * EPW + ch * C, C)
            pltpu.sync_copy(iv_hbm.at[pl.ds(base, C)], iv)
            pltpu.sync_copy(jv_hbm.at[pl.ds(base, C)], jv)
            pltpu.sync_copy(k0_hbm.at[pl.ds(base, C)], k0v)
            pltpu.sync_copy(k1_hbm.at[pl.ds(base, C)], k1v)
            pltpu.sync_copy(k2_hbm.at[pl.ds(base, C)], k2v)

            def j4_body(g, _):
                sl = pl.ds(g * L, L)
                elem4 = (g * (4 * L)) + 4 * lax.iota(jnp.int32, L)
                jj4 = 4 * jv[sl]
                for t in range(4):
                    plsc.store_scatter(jv4, [elem4 + t], jj4 + t)
                return 0

            lax.fori_loop(0, C // L, j4_body, 0)

            cps = [
                pltpu.async_copy(M_hbm.at[iv], Mi_v, sem),
                pltpu.async_copy(Df_hbm.at[jv4], Df_v, sem),
                pltpu.async_copy(M_hbm.at[k0v], Mk0_v, sem),
                pltpu.async_copy(M_hbm.at[k1v], Mk1_v, sem),
                pltpu.async_copy(M_hbm.at[k2v], Mk2_v, sem),
                pltpu.async_copy(mbar_hbm.at[iv], mb_v, sem),
                pltpu.async_copy(dbar_hbm.at[jv], db_v, sem),
            ]
            for cp in cps:
                cp.wait()

            iota = lax.iota(jnp.int32, L)
            iota3 = 3 * iota
            last = iota == (L - 1)

            def elem_body(e, _):
                e4 = 4 * e
                mi = []
                msum = []
                dj = []
                for q in range(R_DIM // L):
                    sl = pl.ds(q * L, L)
                    mi.append(Mi_v[e, sl])
                    msum.append(Mk0_v[e, sl] + Mk1_v[e, sl] + Mk2_v[e, sl])
                    dj.append(Df_v[e4, sl])
                accd = mi[0] * dj[0]
                for q in range(1, R_DIM // L):
                    accd = accd + mi[q] * dj[q]
                res = ALPHA * plsc.cumsum(accd)
                for s in range(S):
                    acca = None
                    accc = None
                    for q in range(R_DIM // L):
                        fa = (R_DIM + s + 48 * q) + iota3
                        va = plsc.load_gather(
                            Df_v, [e4 + (fa >> 7), fa & 127])
                        fc = (R_DIM + S * R_DIM + s + 48 * q) + iota3
                        vc = plsc.load_gather(
                            Df_v, [e4 + (fc >> 7), fc & 127])
                        pa = va * mi[q]
                        pc = vc * msum[q]
                        acca = pa if acca is None else acca + pa
                        accc = pc if accc is None else accc + pc
                    res = res + ((BETA * BETA)
                                 * plsc.cumsum(acca) * plsc.cumsum(accc))
                plsc.store_scatter(out_v, [jnp.full((L,), e, jnp.int32)],
                                   res, mask=last)
                return 0

            lax.fori_loop(0, C, elem_body, 0)

            def fin_body(g, _):
                sl = pl.ds(g * L, L)
                out_v[sl] = out_v[sl] + mb_v[sl] + db_v[sl]
                return 0

            lax.fori_loop(0, C // L, fin_body, 0)
            pltpu.sync_copy(out_v, out_hbm.at[pl.ds(base, C)])
            return 0

        lax.fori_loop(0, NCH, chunk_body, 0)

    m_tab = _tc_pack_m(jnp.swapaxes(M, 0, 1), 4096)
    d_tab = jnp.reshape(_tc_pack_d(jnp.swapaxes(D_full, 0, 1), 1024),
                        (4 * D_full.shape[0], 128))
    ijk = jnp.asarray(ijk, jnp.int32)
    return sc_kernel(ijk[:, 0], ijk[:, 1], ijk[:, 2], ijk[:, 3], ijk[:, 4],
                     m_bar, d_bar, m_tab, d_tab)


# speedup vs baseline: 1.0258x; 1.0258x over previous
import functools

import jax
import jax.numpy as jnp
from jax import lax
from jax.experimental import pallas as pl
from jax.experimental.pallas import tpu as pltpu
from jax.experimental.pallas import tpu_sc as plsc

ALPHA = 0.001
BETA = 0.001
S = 3
R_DIM = 64
DF_DIM = 448
MP = 128
L = 16


def _tc_pack_d(d_t, bs):
    F, N = d_t.shape
    grid = ((N + bs - 1) // bs,)

    def body(i_ref, o_ref):
        for g in range(4):
            w = min(128, F - g * 128)
            o_ref[:, g, 0:w] = i_ref[g * 128:g * 128 + w, :].T

    return pl.pallas_call(
        body,
        grid=grid,
        in_specs=[pl.BlockSpec((F, bs), lambda b: (0, b))],
        out_specs=pl.BlockSpec((bs, 4, 128), lambda b: (b, 0, 0)),
        out_shape=jax.ShapeDtypeStruct((N, 4, 128), d_t.dtype),
    )(d_t)


def _tc_pack_m(m_t, bs):
    F, N = m_t.shape
    grid = ((N + bs - 1) // bs,)

    def body(i_ref, o_ref):
        o_ref[:, 0:F] = i_ref[...].T

    return pl.pallas_call(
        body,
        grid=grid,
        in_specs=[pl.BlockSpec((F, bs), lambda b: (0, b))],
        out_specs=pl.BlockSpec((bs, MP), lambda b: (b, 0)),
        out_shape=jax.ShapeDtypeStruct((N, MP), m_t.dtype),
    )(m_t)


def kernel(ijk, m_bar, d_bar, M, D_full):
    B = ijk.shape[0]
    info = plsc.get_sparse_core_info()
    NC, NS = info.num_cores, info.num_subcores
    NW = NC * NS
    EPW = B // NW
    C = 32
    NCH = EPW // C

    mesh = plsc.VectorSubcoreMesh(core_axis_name="c", subcore_axis_name="s")

    def buf_set():
        return [
            pltpu.VMEM((C,), jnp.int32),
            pltpu.VMEM((C,), jnp.int32),
            pltpu.VMEM((C,), jnp.int32),
            pltpu.VMEM((C,), jnp.int32),
            pltpu.VMEM((C,), jnp.int32),
            pltpu.VMEM((4 * C,), jnp.int32),
            pltpu.VMEM((C,), jnp.float32),
            pltpu.VMEM((C,), jnp.float32),
            pltpu.VMEM((C, MP), jnp.float32),
            pltpu.VMEM((C, MP), jnp.float32),
            pltpu.VMEM((C, MP), jnp.float32),
            pltpu.VMEM((C, MP), jnp.float32),
            pltpu.VMEM((4 * C, 128), jnp.float32),
            pltpu.VMEM((C,), jnp.float32),
            pltpu.SemaphoreType.DMA,
        ]

    @functools.partial(
        pl.kernel,
        mesh=mesh,
        out_type=jax.ShapeDtypeStruct((B,), jnp.float32),
        compiler_params=pltpu.CompilerParams(
            use_tc_tiling_on_sc=False, needs_layout_passes=False),
        scratch_types=buf_set() + buf_set(),
    )
    def sc_kernel(iv_hbm, jv_hbm, k0_hbm, k1_hbm, k2_hbm,
                  mbar_hbm, dbar_hbm, M_hbm, Df_hbm, out_hbm, *bufs):
        set0, set1 = bufs[:15], bufs[15:]
        wid = lax.axis_index("s") * NC + lax.axis_index("c")

        def copies(ch, bufset, make_only):
            (iv, jv, k0v, k1v, k2v, jv4, mb_v, db_v,
             Mi_v, Mk0_v, Mk1_v, Mk2_v, Df_v, out_v, sem) = bufset
            return [
                (M_hbm.at[iv], Mi_v, sem),
                (Df_hbm.at[jv4], Df_v, sem),
                (M_hbm.at[k0v], Mk0_v, sem),
                (M_hbm.at[k1v], Mk1_v, sem),
                (M_hbm.at[k2v], Mk2_v, sem),
                (mbar_hbm.at[iv], mb_v, sem),
                (dbar_hbm.at[jv], db_v, sem),
            ]

        def fire(ch, bufset):
            (iv, jv, k0v, k1v, k2v, jv4, mb_v, db_v,
             Mi_v, Mk0_v, Mk1_v, Mk2_v, Df_v, out_v, sem) = bufset
            base = pl.multiple_of(wid * EPW + ch * C, C)
            pltpu.sync_copy(iv_hbm.at[pl.ds(base, C)], iv)
            pltpu.sync_copy(jv_hbm.at[pl.ds(base, C)], jv)
            pltpu.sync_copy(k0_hbm.at[pl.ds(base, C)], k0v)
            pltpu.sync_copy(k1_hbm.at[pl.ds(base, C)], k1v)
            pltpu.sync_copy(k2_hbm.at[pl.ds(base, C)], k2v)

            def j4_body(g, _):
                sl = pl.ds(g * L, L)
                elem4 = (g * (4 * L)) + 4 * lax.iota(jnp.int32, L)
                jj4 = 4 * jv[sl]
                for t in range(4):
                    plsc.store_scatter(jv4, [elem4 + t], jj4 + t)
                return 0

            lax.fori_loop(0, C // L, j4_body, 0)
            for src, dst, sem_ in copies(ch, bufset, False):
                pltpu.async_copy(src, dst, sem_)

        def drain(bufset):
            for src, dst, sem_ in copies(0, bufset, True):
                pltpu.make_async_copy(src, dst, sem_).wait()

        def compute(ch, bufset):
            (iv, jv, k0v, k1v, k2v, jv4, mb_v, db_v,
             Mi_v, Mk0_v, Mk1_v, Mk2_v, Df_v, out_v, sem) = bufset
            base = pl.multiple_of(wid * EPW + ch * C, C)
            iota = lax.iota(jnp.int32, L)
            iota3 = 3 * iota
            last = iota == (L - 1)

            def elem_body(e, _):
                e4 = 4 * e
                mi = []
                msum = []
                dj = []
                for q in range(R_DIM // L):
                    sl = pl.ds(q * L, L)
                    mi.append(Mi_v[e, sl])
                    msum.append(Mk0_v[e, sl] + Mk1_v[e, sl] + Mk2_v[e, sl])
                    dj.append(Df_v[e4, sl])
                accd = mi[0] * dj[0]
                for q in range(1, R_DIM // L):
                    accd = accd + mi[q] * dj[q]
                res = ALPHA * plsc.cumsum(accd)
                for s in range(S):
                    acca = None
                    accc = None
                    for q in range(R_DIM // L):
                        fa = (R_DIM + s + 48 * q) + iota3
                        va = plsc.load_gather(
                            Df_v, [e4 + (fa >> 7), fa & 127])
                        fc = (R_DIM + S * R_DIM + s + 48 * q) + iota3
                        vc = plsc.load_gather(
                            Df_v, [e4 + (fc >> 7), fc & 127])
                        pa = va * mi[q]
                        pc = vc * msum[q]
                        acca = pa if acca is None else acca + pa
                        accc = pc if accc is None else accc + pc
                    res = res + ((BETA * BETA)
                                 * plsc.cumsum(acca) * plsc.cumsum(accc))
                plsc.store_scatter(out_v, [jnp.full((L,), e, jnp.int32)],
                                   res, mask=last)
                return 0

            lax.fori_loop(0, C, elem_body, 0)

            def fin_body(g, _):
                sl = pl.ds(g * L, L)
                out_v[sl] = out_v[sl] + mb_v[sl] + db_v[sl]
                return 0

            lax.fori_loop(0, C // L, fin_body, 0)
            pltpu.sync_copy(out_v, out_hbm.at[pl.ds(base, C)])

        fire(0, set0)

        def pair_body(p, _):
            ch0 = 2 * p
            fire(ch0 + 1, set1)
            drain(set0)
            compute(ch0, set0)

            @pl.when(ch0 + 2 < NCH)
            def _():
                fire(ch0 + 2, set0)

            drain(set1)
            compute(ch0 + 1, set1)
            return 0

        lax.fori_loop(0, NCH // 2, pair_body, 0)

    m_tab = _tc_pack_m(jnp.swapaxes(M, 0, 1), 4096)
    d_tab = jnp.reshape(_tc_pack_d(jnp.swapaxes(D_full, 0, 1), 1024),
                        (4 * D_full.shape[0], 128))
    ijk = jnp.asarray(ijk, jnp.int32)
    return sc_kernel(ijk[:, 0], ijk[:, 1], ijk[:, 2], ijk[:, 3], ijk[:, 4],
                     m_bar, d_bar, m_tab, d_tab)


# speedup vs baseline: 1.1296x; 1.1011x over previous
import functools

import jax
import jax.numpy as jnp
from jax import lax
from jax.experimental import pallas as pl
from jax.experimental.pallas import tpu as pltpu
from jax.experimental.pallas import tpu_sc as plsc

ALPHA = 0.001
BETA = 0.001
S = 3
R_DIM = 64
DF_DIM = 448
MP = 128
L = 16


def _tc_pack_d(d_t, bs):
    F, N = d_t.shape
    grid = ((N + bs - 1) // bs,)

    def body(i_ref, o_ref):
        for g in range(4):
            w = min(128, F - g * 128)
            o_ref[:, g, 0:w] = i_ref[g * 128:g * 128 + w, :].T

    return pl.pallas_call(
        body,
        grid=grid,
        in_specs=[pl.BlockSpec((F, bs), lambda b: (0, b))],
        out_specs=pl.BlockSpec((bs, 4, 128), lambda b: (b, 0, 0)),
        out_shape=jax.ShapeDtypeStruct((N, 4, 128), d_t.dtype),
    )(d_t)


def _tc_pack_m(m_t, bs):
    F, N = m_t.shape
    grid = ((N + bs - 1) // bs,)

    def body(i_ref, o_ref):
        o_ref[:, 0:F] = i_ref[...].T

    return pl.pallas_call(
        body,
        grid=grid,
        in_specs=[pl.BlockSpec((F, bs), lambda b: (0, b))],
        out_specs=pl.BlockSpec((bs, MP), lambda b: (b, 0)),
        out_shape=jax.ShapeDtypeStruct((N, MP), m_t.dtype),
    )(m_t)


def kernel(ijk, m_bar, d_bar, M, D_full):
    B = ijk.shape[0]
    info = plsc.get_sparse_core_info()
    NC, NS = info.num_cores, info.num_subcores
    NW = NC * NS
    EPW = B // NW
    C = 32
    NCH = EPW // C

    mesh = plsc.VectorSubcoreMesh(core_axis_name="c", subcore_axis_name="s")

    def buf_set():
        return [
            pltpu.VMEM((C, MP), jnp.float32),
            pltpu.VMEM((C, MP), jnp.float32),
            pltpu.VMEM((C, MP), jnp.float32),
            pltpu.VMEM((C, MP), jnp.float32),
            pltpu.VMEM((4 * C, 128), jnp.float32),
            pltpu.VMEM((C,), jnp.float32),
            pltpu.SemaphoreType.DMA,
        ]

    @functools.partial(
        pl.kernel,
        mesh=mesh,
        out_type=jax.ShapeDtypeStruct((B,), jnp.float32),
        compiler_params=pltpu.CompilerParams(
            use_tc_tiling_on_sc=False, needs_layout_passes=False),
        scratch_types=[
            pltpu.VMEM((EPW,), jnp.int32),
            pltpu.VMEM((EPW,), jnp.int32),
            pltpu.VMEM((EPW,), jnp.int32),
            pltpu.VMEM((EPW,), jnp.int32),
            pltpu.VMEM((EPW,), jnp.int32),
            pltpu.VMEM((4 * EPW,), jnp.int32),
            pltpu.VMEM((EPW,), jnp.float32),
            pltpu.VMEM((EPW,), jnp.float32),
            pltpu.SemaphoreType.DMA,
        ] + buf_set() + buf_set(),
    )
    def sc_kernel(iv_hbm, jv_hbm, k0_hbm, k1_hbm, k2_hbm,
                  mbar_hbm, dbar_hbm, M_hbm, Df_hbm, out_hbm,
                  iv_all, jv_all, k0_all, k1_all, k2_all, jv4_all,
                  mb_all, db_all, sem_pre, *bufs):
        set0, set1 = bufs[:7], bufs[7:]
        wid = lax.axis_index("s") * NC + lax.axis_index("c")
        wbase = pl.multiple_of(wid * EPW, EPW)

        pltpu.sync_copy(iv_hbm.at[pl.ds(wbase, EPW)], iv_all)
        pltpu.sync_copy(jv_hbm.at[pl.ds(wbase, EPW)], jv_all)
        pltpu.sync_copy(k0_hbm.at[pl.ds(wbase, EPW)], k0_all)
        pltpu.sync_copy(k1_hbm.at[pl.ds(wbase, EPW)], k1_all)
        pltpu.sync_copy(k2_hbm.at[pl.ds(wbase, EPW)], k2_all)

        def j4_body(g, _):
            sl = pl.ds(g * L, L)
            elem4 = (g * (4 * L)) + 4 * lax.iota(jnp.int32, L)
            jj4 = 4 * jv_all[sl]
            for t in range(4):
                plsc.store_scatter(jv4_all, [elem4 + t], jj4 + t)
            return 0

        lax.fori_loop(0, EPW // L, j4_body, 0)
        pltpu.async_copy(mbar_hbm.at[iv_all], mb_all, sem_pre)
        pltpu.async_copy(dbar_hbm.at[jv_all], db_all, sem_pre)

        def copies(ch, bufset):
            Mi_v, Mk0_v, Mk1_v, Mk2_v, Df_v, out_v, sem = bufset
            co = ch * C
            return [
                (M_hbm.at[iv_all.at[pl.ds(co, C)]], Mi_v, sem),
                (Df_hbm.at[jv4_all.at[pl.ds(4 * co, 4 * C)]], Df_v, sem),
                (M_hbm.at[k0_all.at[pl.ds(co, C)]], Mk0_v, sem),
                (M_hbm.at[k1_all.at[pl.ds(co, C)]], Mk1_v, sem),
                (M_hbm.at[k2_all.at[pl.ds(co, C)]], Mk2_v, sem),
            ]

        def fire(ch, bufset):
            for src, dst, sem_ in copies(ch, bufset):
                pltpu.async_copy(src, dst, sem_)

        def drain(bufset):
            for src, dst, sem_ in copies(0, bufset):
                pltpu.make_async_copy(src, dst, sem_).wait()

        def compute(ch, bufset):
            Mi_v, Mk0_v, Mk1_v, Mk2_v, Df_v, out_v, sem = bufset
            base = pl.multiple_of(wid * EPW + ch * C, C)
            iota = lax.iota(jnp.int32, L)
            iota3 = 3 * iota
            last = iota == (L - 1)

            def elem_body(e, _):
                e4 = 4 * e
                mi = []
                msum = []
                dj = []
                for q in range(R_DIM // L):
                    sl = pl.ds(q * L, L)
                    mi.append(Mi_v[e, sl])
                    msum.append(Mk0_v[e, sl] + Mk1_v[e, sl] + Mk2_v[e, sl])
                    dj.append(Df_v[e4, sl])
                accd = mi[0] * dj[0]
                for q in range(1, R_DIM // L):
                    accd = accd + mi[q] * dj[q]
                res = ALPHA * plsc.cumsum(accd)
                for s in range(S):
                    acca = None
                    accc = None
                    for q in range(R_DIM // L):
                        fa = (R_DIM + s + 48 * q) + iota3
                        va = plsc.load_gather(
                            Df_v, [e4 + (fa >> 7), fa & 127])
                        fc = (R_DIM + S * R_DIM + s + 48 * q) + iota3
                        vc = plsc.load_gather(
                            Df_v, [e4 + (fc >> 7), fc & 127])
                        pa = va * mi[q]
                        pc = vc * msum[q]
                        acca = pa if acca is None else acca + pa
                        accc = pc if accc is None else accc + pc
                    res = res + ((BETA * BETA)
                                 * plsc.cumsum(acca) * plsc.cumsum(accc))
                plsc.store_scatter(out_v, [jnp.full((L,), e, jnp.int32)],
                                   res, mask=last)
                return 0

            lax.fori_loop(0, C, elem_body, 0)

            def fin_body(g, _):
                sl = pl.ds(g * L, L)
                sla = pl.ds(ch * C + g * L, L)
                out_v[sl] = out_v[sl] + mb_all[sla] + db_all[sla]
                return 0

            lax.fori_loop(0, C // L, fin_body, 0)
            pltpu.sync_copy(out_v, out_hbm.at[pl.ds(base, C)])

        fire(0, set0)
        pltpu.make_async_copy(mbar_hbm.at[iv_all], mb_all, sem_pre).wait()
        pltpu.make_async_copy(dbar_hbm.at[jv_all], db_all, sem_pre).wait()

        def pair_body(p, _):
            ch0 = 2 * p
            fire(ch0 + 1, set1)
            drain(set0)
            compute(ch0, set0)

            @pl.when(ch0 + 2 < NCH)
            def _():
                fire(ch0 + 2, set0)

            drain(set1)
            compute(ch0 + 1, set1)
            return 0

        lax.fori_loop(0, NCH // 2, pair_body, 0)

    m_tab = _tc_pack_m(jnp.swapaxes(M, 0, 1), 4096)
    d_tab = jnp.reshape(_tc_pack_d(jnp.swapaxes(D_full, 0, 1), 1024),
                        (4 * D_full.shape[0], 128))
    ijk = jnp.asarray(ijk, jnp.int32)
    return sc_kernel(ijk[:, 0], ijk[:, 1], ijk[:, 2], ijk[:, 3], ijk[:, 4],
                     m_bar, d_bar, m_tab, d_tab)


# speedup vs baseline: 1.2972x; 1.1483x over previous
import functools

import jax
import jax.numpy as jnp
from jax import lax
from jax.experimental import pallas as pl
from jax.experimental.pallas import tpu as pltpu
from jax.experimental.pallas import tpu_sc as plsc

ALPHA = 0.001
BETA = 0.001
S = 3
R_DIM = 64
DF_DIM = 448
MP = 128
L = 16


def _tc_pack_d(d_t, bs):
    F, N = d_t.shape
    grid = ((N + bs - 1) // bs,)

    def body(i_ref, o_ref):
        for g in range(4):
            w = min(128, F - g * 128)
            o_ref[:, g, 0:w] = i_ref[g * 128:g * 128 + w, :].T

    return pl.pallas_call(
        body,
        grid=grid,
        in_specs=[pl.BlockSpec((F, bs), lambda b: (0, b))],
        out_specs=pl.BlockSpec((bs, 4, 128), lambda b: (b, 0, 0)),
        out_shape=jax.ShapeDtypeStruct((N, 4, 128), d_t.dtype),
    )(d_t)


def _tc_pack_m(m_t, bs):
    F, N = m_t.shape
    grid = ((N + bs - 1) // bs,)

    def body(i_ref, o_ref):
        o_ref[:, 0:F] = i_ref[...].T

    return pl.pallas_call(
        body,
        grid=grid,
        in_specs=[pl.BlockSpec((F, bs), lambda b: (0, b))],
        out_specs=pl.BlockSpec((bs, MP), lambda b: (b, 0)),
        out_shape=jax.ShapeDtypeStruct((N, MP), m_t.dtype),
    )(m_t)


def kernel(ijk, m_bar, d_bar, M, D_full):
    B = ijk.shape[0]
    info = plsc.get_sparse_core_info()
    NC, NS = info.num_cores, info.num_subcores
    NW = NC * NS
    EPW = B // NW
    C = 32
    NCH = EPW // C

    mesh = plsc.VectorSubcoreMesh(core_axis_name="c", subcore_axis_name="s")

    def buf_set():
        return [
            pltpu.VMEM((C, MP), jnp.float32),
            pltpu.VMEM((C, MP), jnp.float32),
            pltpu.VMEM((C, MP), jnp.float32),
            pltpu.VMEM((C, MP), jnp.float32),
            pltpu.VMEM((4 * C, 128), jnp.float32),
            pltpu.VMEM((C,), jnp.float32),
            pltpu.SemaphoreType.DMA,
        ]

    @functools.partial(
        pl.kernel,
        mesh=mesh,
        out_type=jax.ShapeDtypeStruct((B,), jnp.float32),
        compiler_params=pltpu.CompilerParams(
            use_tc_tiling_on_sc=False, needs_layout_passes=False),
        scratch_types=[
            pltpu.VMEM((EPW,), jnp.int32),
            pltpu.VMEM((EPW,), jnp.int32),
            pltpu.VMEM((EPW,), jnp.int32),
            pltpu.VMEM((EPW,), jnp.int32),
            pltpu.VMEM((EPW,), jnp.int32),
            pltpu.VMEM((4 * EPW,), jnp.int32),
            pltpu.VMEM((EPW,), jnp.float32),
            pltpu.VMEM((EPW,), jnp.float32),
            pltpu.SemaphoreType.DMA,
        ] + buf_set() + buf_set(),
    )
    def sc_kernel(iv_hbm, jv_hbm, k0_hbm, k1_hbm, k2_hbm,
                  mbar_hbm, dbar_hbm, M_hbm, Df_hbm, out_hbm,
                  iv_all, jv_all, k0_all, k1_all, k2_all, jv4_all,
                  mb_all, db_all, sem_pre, *bufs):
        set0, set1 = bufs[:7], bufs[7:]
        wid = lax.axis_index("s") * NC + lax.axis_index("c")
        wbase = pl.multiple_of(wid * EPW, EPW)

        pltpu.sync_copy(iv_hbm.at[pl.ds(wbase, EPW)], iv_all)
        pltpu.sync_copy(jv_hbm.at[pl.ds(wbase, EPW)], jv_all)
        pltpu.sync_copy(k0_hbm.at[pl.ds(wbase, EPW)], k0_all)
        pltpu.sync_copy(k1_hbm.at[pl.ds(wbase, EPW)], k1_all)
        pltpu.sync_copy(k2_hbm.at[pl.ds(wbase, EPW)], k2_all)

        def j4_body(g, _):
            sl = pl.ds(g * L, L)
            elem4 = (g * (4 * L)) + 4 * lax.iota(jnp.int32, L)
            jj4 = 4 * jv_all[sl]
            for t in range(4):
                plsc.store_scatter(jv4_all, [elem4 + t], jj4 + t)
            return 0

        lax.fori_loop(0, EPW // L, j4_body, 0)
        pltpu.async_copy(mbar_hbm.at[iv_all], mb_all, sem_pre)
        pltpu.async_copy(dbar_hbm.at[jv_all], db_all, sem_pre)

        def copies(ch, bufset):
            Mi_v, Mk0_v, Mk1_v, Mk2_v, Df_v, out_v, sem = bufset
            co = ch * C
            return [
                (M_hbm.at[iv_all.at[pl.ds(co, C)]], Mi_v, sem),
                (Df_hbm.at[jv4_all.at[pl.ds(4 * co, 4 * C)]], Df_v, sem),
                (M_hbm.at[k0_all.at[pl.ds(co, C)]], Mk0_v, sem),
                (M_hbm.at[k1_all.at[pl.ds(co, C)]], Mk1_v, sem),
                (M_hbm.at[k2_all.at[pl.ds(co, C)]], Mk2_v, sem),
            ]

        def fire(ch, bufset):
            for src, dst, sem_ in copies(ch, bufset):
                pltpu.async_copy(src, dst, sem_)

        def drain(bufset):
            for src, dst, sem_ in copies(0, bufset):
                pltpu.make_async_copy(src, dst, sem_).wait()

        def compute(ch, bufset):
            Mi_v, Mk0_v, Mk1_v, Mk2_v, Df_v, out_v, sem = bufset
            base = pl.multiple_of(wid * EPW + ch * C, C)
            iota = lax.iota(jnp.int32, L)
            iota3 = 3 * iota
            last = iota == (L - 1)

            def elem_body(e, _):
                e4 = 4 * e
                mi = []
                msum = []
                dj = []
                for q in range(R_DIM // L):
                    sl = pl.ds(q * L, L)
                    mi.append(Mi_v[e, sl])
                    msum.append(Mk0_v[e, sl] + Mk1_v[e, sl] + Mk2_v[e, sl])
                    dj.append(Df_v[e4, sl])
                accd = mi[0] * dj[0]
                for q in range(1, R_DIM // L):
                    accd = accd + mi[q] * dj[q]
                res = ALPHA * plsc.cumsum(accd)
                for s in range(S):
                    acca = None
                    accc = None
                    for q in range(R_DIM // L):
                        fa = (R_DIM + s + 48 * q) + iota3
                        va = plsc.load_gather(
                            Df_v, [e4 + (fa >> 7), fa & 127])
                        fc = (R_DIM + S * R_DIM + s + 48 * q) + iota3
                        vc = plsc.load_gather(
                            Df_v, [e4 + (fc >> 7), fc & 127])
                        pa = va * mi[q]
                        pc = vc * msum[q]
                        acca = pa if acca is None else acca + pa
                        accc = pc if accc is None else accc + pc
                    res = res + ((BETA * BETA)
                                 * plsc.cumsum(acca) * plsc.cumsum(accc))
                plsc.store_scatter(out_v, [jnp.full((L,), e, jnp.int32)],
                                   res, mask=last)
                return 0

            lax.fori_loop(0, C, elem_body, 0)

            def fin_body(g, _):
                sl = pl.ds(g * L, L)
                sla = pl.ds(ch * C + g * L, L)
                out_v[sl] = out_v[sl] + mb_all[sla] + db_all[sla]
                return 0

            lax.fori_loop(0, C // L, fin_body, 0)
            pltpu.sync_copy(out_v, out_hbm.at[pl.ds(base, C)])

        fire(0, set0)
        pltpu.make_async_copy(mbar_hbm.at[iv_all], mb_all, sem_pre).wait()
        pltpu.make_async_copy(dbar_hbm.at[jv_all], db_all, sem_pre).wait()

        def pair_body(p, _):
            ch0 = 2 * p
            fire(ch0 + 1, set1)
            drain(set0)
            compute(ch0, set0)

            @pl.when(ch0 + 2 < NCH)
            def _():
                fire(ch0 + 2, set0)

            drain(set1)
            compute(ch0 + 1, set1)
            return 0

        lax.fori_loop(0, NCH // 2, pair_body, 0)

    m_tab = _tc_pack_m(jnp.swapaxes(M, 0, 1), 8192)
    d_tab = jnp.reshape(_tc_pack_d(jnp.swapaxes(D_full, 0, 1), 2048),
                        (4 * D_full.shape[0], 128))
    ijk = jnp.asarray(ijk, jnp.int32)
    return sc_kernel(ijk[:, 0], ijk[:, 1], ijk[:, 2], ijk[:, 3], ijk[:, 4],
                     m_bar, d_bar, m_tab, d_tab)


# speedup vs baseline: 1.3814x; 1.0650x over previous
import functools

import jax
import jax.numpy as jnp
from jax import lax
from jax.experimental import pallas as pl
from jax.experimental.pallas import tpu as pltpu
from jax.experimental.pallas import tpu_sc as plsc

ALPHA = 0.001
BETA = 0.001
S = 3
R_DIM = 64
DF_DIM = 448
MP = 128
L = 16


def _tc_pack_d(d_t, bs):
    F, N = d_t.shape
    grid = ((N + bs - 1) // bs,)

    def body(i_ref, o_ref):
        for g in range(4):
            w = min(128, F - g * 128)
            o_ref[:, g, 0:w] = i_ref[g * 128:g * 128 + w, :].T

    return pl.pallas_call(
        body,
        grid=grid,
        in_specs=[pl.BlockSpec((F, bs), lambda b: (0, b))],
        out_specs=pl.BlockSpec((bs, 4, 128), lambda b: (b, 0, 0)),
        out_shape=jax.ShapeDtypeStruct((N, 4, 128), d_t.dtype),
    )(d_t)


def _tc_pack_m(m_t, bs):
    F, N = m_t.shape
    grid = ((N + bs - 1) // bs,)

    def body(i_ref, o_ref):
        o_ref[:, 0:F] = i_ref[...].T

    return pl.pallas_call(
        body,
        grid=grid,
        in_specs=[pl.BlockSpec((F, bs), lambda b: (0, b))],
        out_specs=pl.BlockSpec((bs, MP), lambda b: (b, 0)),
        out_shape=jax.ShapeDtypeStruct((N, MP), m_t.dtype),
    )(m_t)


def kernel(ijk, m_bar, d_bar, M, D_full):
    B = ijk.shape[0]
    info = plsc.get_sparse_core_info()
    NC, NS = info.num_cores, info.num_subcores
    NW = NC * NS
    EPW = B // NW
    C = 32
    NCH = EPW // C

    mesh = plsc.VectorSubcoreMesh(core_axis_name="c", subcore_axis_name="s")

    def buf_set():
        return [
            pltpu.VMEM((C, MP), jnp.float32),
            pltpu.VMEM((C, MP), jnp.float32),
            pltpu.VMEM((C, MP), jnp.float32),
            pltpu.VMEM((C, MP), jnp.float32),
            pltpu.VMEM((4 * C, 128), jnp.float32),
            pltpu.VMEM((C,), jnp.float32),
            pltpu.SemaphoreType.DMA,
        ]

    @functools.partial(
        pl.kernel,
        mesh=mesh,
        out_type=jax.ShapeDtypeStruct((B,), jnp.float32),
        compiler_params=pltpu.CompilerParams(
            use_tc_tiling_on_sc=False, needs_layout_passes=False),
        scratch_types=[
            pltpu.VMEM((EPW,), jnp.int32),
            pltpu.VMEM((EPW,), jnp.int32),
            pltpu.VMEM((EPW,), jnp.int32),
            pltpu.VMEM((EPW,), jnp.int32),
            pltpu.VMEM((EPW,), jnp.int32),
            pltpu.VMEM((4 * EPW,), jnp.int32),
            pltpu.VMEM((EPW,), jnp.float32),
            pltpu.VMEM((EPW,), jnp.float32),
            pltpu.SemaphoreType.DMA,
        ] + buf_set() + buf_set(),
    )
    def sc_kernel(iv_hbm, jv_hbm, k0_hbm, k1_hbm, k2_hbm,
                  mbar_hbm, dbar_hbm, M_hbm, Df_hbm, out_hbm,
                  iv_all, jv_all, k0_all, k1_all, k2_all, jv4_all,
                  mb_all, db_all, sem_pre, *bufs):
        set0, set1 = bufs[:7], bufs[7:]
        wid = lax.axis_index("s") * NC + lax.axis_index("c")
        wbase = pl.multiple_of(wid * EPW, EPW)

        pltpu.sync_copy(iv_hbm.at[pl.ds(wbase, EPW)], iv_all)
        pltpu.sync_copy(jv_hbm.at[pl.ds(wbase, EPW)], jv_all)
        pltpu.sync_copy(k0_hbm.at[pl.ds(wbase, EPW)], k0_all)
        pltpu.sync_copy(k1_hbm.at[pl.ds(wbase, EPW)], k1_all)
        pltpu.sync_copy(k2_hbm.at[pl.ds(wbase, EPW)], k2_all)

        def j4_body(g, _):
            sl = pl.ds(g * L, L)
            elem4 = (g * (4 * L)) + 4 * lax.iota(jnp.int32, L)
            jj4 = 4 * jv_all[sl]
            for t in range(4):
                plsc.store_scatter(jv4_all, [elem4 + t], jj4 + t)
            return 0

        lax.fori_loop(0, EPW // L, j4_body, 0)
        pltpu.async_copy(mbar_hbm.at[iv_all], mb_all, sem_pre)
        pltpu.async_copy(dbar_hbm.at[jv_all], db_all, sem_pre)

        def copies(ch, bufset):
            Mi_v, Mk0_v, Mk1_v, Mk2_v, Df_v, out_v, sem = bufset
            co = ch * C
            return [
                (M_hbm.at[iv_all.at[pl.ds(co, C)]], Mi_v, sem),
                (Df_hbm.at[jv4_all.at[pl.ds(4 * co, 4 * C)]], Df_v, sem),
                (M_hbm.at[k0_all.at[pl.ds(co, C)]], Mk0_v, sem),
                (M_hbm.at[k1_all.at[pl.ds(co, C)]], Mk1_v, sem),
                (M_hbm.at[k2_all.at[pl.ds(co, C)]], Mk2_v, sem),
            ]

        def fire(ch, bufset):
            for src, dst, sem_ in copies(ch, bufset):
                pltpu.async_copy(src, dst, sem_)

        def drain(bufset):
            for src, dst, sem_ in copies(0, bufset):
                pltpu.make_async_copy(src, dst, sem_).wait()

        def compute(ch, bufset):
            Mi_v, Mk0_v, Mk1_v, Mk2_v, Df_v, out_v, sem = bufset
            base = pl.multiple_of(wid * EPW + ch * C, C)
            iota = lax.iota(jnp.int32, L)
            iota3 = 3 * iota
            last = iota == (L - 1)

            def elem_body(e, _):
                e4 = 4 * e
                mi = []
                msum = []
                dj = []
                for q in range(R_DIM // L):
                    sl = pl.ds(q * L, L)
                    mi.append(Mi_v[e, sl])
                    msum.append(Mk0_v[e, sl] + Mk1_v[e, sl] + Mk2_v[e, sl])
                    dj.append(Df_v[e4, sl])
                accd = mi[0] * dj[0]
                for q in range(1, R_DIM // L):
                    accd = accd + mi[q] * dj[q]
                res = ALPHA * plsc.cumsum(accd)
                for s in range(S):
                    acca = None
                    accc = None
                    for q in range(R_DIM // L):
                        fa = (R_DIM + s + 48 * q) + iota3
                        va = plsc.load_gather(
                            Df_v, [e4 + (fa >> 7), fa & 127])
                        fc = (R_DIM + S * R_DIM + s + 48 * q) + iota3
                        vc = plsc.load_gather(
                            Df_v, [e4 + (fc >> 7), fc & 127])
                        pa = va * mi[q]
                        pc = vc * msum[q]
                        acca = pa if acca is None else acca + pa
                        accc = pc if accc is None else accc + pc
                    res = res + ((BETA * BETA)
                                 * plsc.cumsum(acca) * plsc.cumsum(accc))
                plsc.store_scatter(out_v, [jnp.full((L,), e, jnp.int32)],
                                   res, mask=last)
                return 0

            lax.fori_loop(0, C, elem_body, 0)

            def fin_body(g, _):
                sl = pl.ds(g * L, L)
                sla = pl.ds(ch * C + g * L, L)
                out_v[sl] = out_v[sl] + mb_all[sla] + db_all[sla]
                return 0

            lax.fori_loop(0, C // L, fin_body, 0)
            pltpu.sync_copy(out_v, out_hbm.at[pl.ds(base, C)])

        fire(0, set0)
        pltpu.make_async_copy(mbar_hbm.at[iv_all], mb_all, sem_pre).wait()
        pltpu.make_async_copy(dbar_hbm.at[jv_all], db_all, sem_pre).wait()

        def pair_body(p, _):
            ch0 = 2 * p
            fire(ch0 + 1, set1)
            drain(set0)
            compute(ch0, set0)

            @pl.when(ch0 + 2 < NCH)
            def _():
                fire(ch0 + 2, set0)

            drain(set1)
            compute(ch0 + 1, set1)
            return 0

        lax.fori_loop(0, NCH // 2, pair_body, 0)

    m_tab = _tc_pack_m(jnp.swapaxes(M, 0, 1), 16384)
    d_tab = jnp.reshape(_tc_pack_d(jnp.swapaxes(D_full, 0, 1), 4096),
                        (4 * D_full.shape[0], 128))
    ijk = jnp.asarray(ijk, jnp.int32)
    return sc_kernel(ijk[:, 0], ijk[:, 1], ijk[:, 2], ijk[:, 3], ijk[:, 4],
                     m_bar, d_bar, m_tab, d_tab)


# speedup vs baseline: 1.3976x; 1.0117x over previous
import functools

import jax
import jax.numpy as jnp
from jax import lax
from jax.experimental import pallas as pl
from jax.experimental.pallas import tpu as pltpu
from jax.experimental.pallas import tpu_sc as plsc

ALPHA = 0.001
BETA = 0.001
S = 3
R_DIM = 64
DF_DIM = 448
MP = 128
L = 16


def _tc_pack_d(d_t, bs):
    F, N = d_t.shape
    grid = ((N + bs - 1) // bs,)

    def body(i_ref, o_ref):
        for g in range(4):
            w = min(128, F - g * 128)
            o_ref[:, g, 0:w] = i_ref[g * 128:g * 128 + w, :].T

    return pl.pallas_call(
        body,
        grid=grid,
        in_specs=[pl.BlockSpec((F, bs), lambda b: (0, b))],
        out_specs=pl.BlockSpec((bs, 4, 128), lambda b: (b, 0, 0)),
        out_shape=jax.ShapeDtypeStruct((N, 4, 128), d_t.dtype),
    )(d_t)


def _tc_pack_m(m_t, bs):
    F, N = m_t.shape
    grid = ((N + bs - 1) // bs,)

    def body(i_ref, o_ref):
        o_ref[:, 0:F] = i_ref[...].T

    return pl.pallas_call(
        body,
        grid=grid,
        in_specs=[pl.BlockSpec((F, bs), lambda b: (0, b))],
        out_specs=pl.BlockSpec((bs, MP), lambda b: (b, 0)),
        out_shape=jax.ShapeDtypeStruct((N, MP), m_t.dtype),
    )(m_t)


def kernel(ijk, m_bar, d_bar, M, D_full):
    B = ijk.shape[0]
    info = plsc.get_sparse_core_info()
    NC, NS = info.num_cores, info.num_subcores
    NW = NC * NS
    EPW = B // NW
    C = 32
    NCH = EPW // C

    mesh = plsc.VectorSubcoreMesh(core_axis_name="c", subcore_axis_name="s")

    def buf_set():
        return [
            pltpu.VMEM((C, MP), jnp.float32),
            pltpu.VMEM((C, MP), jnp.float32),
            pltpu.VMEM((C, MP), jnp.float32),
            pltpu.VMEM((C, MP), jnp.float32),
            pltpu.VMEM((4 * C, 128), jnp.float32),
            pltpu.VMEM((C,), jnp.float32),
            pltpu.SemaphoreType.DMA,
        ]

    @functools.partial(
        pl.kernel,
        mesh=mesh,
        out_type=jax.ShapeDtypeStruct((B,), jnp.float32),
        compiler_params=pltpu.CompilerParams(
            use_tc_tiling_on_sc=False, needs_layout_passes=False),
        scratch_types=[
            pltpu.VMEM((EPW,), jnp.int32),
            pltpu.VMEM((EPW,), jnp.int32),
            pltpu.VMEM((EPW,), jnp.int32),
            pltpu.VMEM((EPW,), jnp.int32),
            pltpu.VMEM((EPW,), jnp.int32),
            pltpu.VMEM((4 * EPW,), jnp.int32),
            pltpu.VMEM((EPW,), jnp.float32),
            pltpu.VMEM((EPW,), jnp.float32),
            pltpu.SemaphoreType.DMA,
        ] + buf_set() + buf_set(),
    )
    def sc_kernel(iv_hbm, jv_hbm, k0_hbm, k1_hbm, k2_hbm,
                  mbar_hbm, dbar_hbm, M_hbm, Df_hbm, out_hbm,
                  iv_all, jv_all, k0_all, k1_all, k2_all, jv4_all,
                  mb_all, db_all, sem_pre, *bufs):
        set0, set1 = bufs[:7], bufs[7:]
        wid = lax.axis_index("s") * NC + lax.axis_index("c")
        wbase = pl.multiple_of(wid * EPW, EPW)

        pltpu.sync_copy(iv_hbm.at[pl.ds(wbase, EPW)], iv_all)
        pltpu.sync_copy(jv_hbm.at[pl.ds(wbase, EPW)], jv_all)
        pltpu.sync_copy(k0_hbm.at[pl.ds(wbase, EPW)], k0_all)
        pltpu.sync_copy(k1_hbm.at[pl.ds(wbase, EPW)], k1_all)
        pltpu.sync_copy(k2_hbm.at[pl.ds(wbase, EPW)], k2_all)

        def j4_body(g, _):
            sl = pl.ds(g * L, L)
            elem4 = (g * (4 * L)) + 4 * lax.iota(jnp.int32, L)
            jj4 = 4 * jv_all[sl]
            for t in range(4):
                plsc.store_scatter(jv4_all, [elem4 + t], jj4 + t)
            return 0

        lax.fori_loop(0, EPW // L, j4_body, 0)
        pltpu.async_copy(mbar_hbm.at[iv_all], mb_all, sem_pre)
        pltpu.async_copy(dbar_hbm.at[jv_all], db_all, sem_pre)

        def copies(ch, bufset):
            Mi_v, Mk0_v, Mk1_v, Mk2_v, Df_v, out_v, sem = bufset
            co = ch * C
            return [
                (M_hbm.at[iv_all.at[pl.ds(co, C)]], Mi_v, sem),
                (Df_hbm.at[jv4_all.at[pl.ds(4 * co, 4 * C)]], Df_v, sem),
                (M_hbm.at[k0_all.at[pl.ds(co, C)]], Mk0_v, sem),
                (M_hbm.at[k1_all.at[pl.ds(co, C)]], Mk1_v, sem),
                (M_hbm.at[k2_all.at[pl.ds(co, C)]], Mk2_v, sem),
            ]

        def fire(ch, bufset):
            for src, dst, sem_ in copies(ch, bufset):
                pltpu.async_copy(src, dst, sem_)

        def drain(bufset):
            for src, dst, sem_ in copies(0, bufset):
                pltpu.make_async_copy(src, dst, sem_).wait()

        def compute(ch, bufset):
            Mi_v, Mk0_v, Mk1_v, Mk2_v, Df_v, out_v, sem = bufset
            base = pl.multiple_of(wid * EPW + ch * C, C)
            iota = lax.iota(jnp.int32, L)
            iota3 = 3 * iota
            last = iota == (L - 1)

            def elem_body(e, _):
                e4 = 4 * e
                mi = []
                msum = []
                dj = []
                for q in range(R_DIM // L):
                    sl = pl.ds(q * L, L)
                    mi.append(Mi_v[e, sl])
                    msum.append(Mk0_v[e, sl] + Mk1_v[e, sl] + Mk2_v[e, sl])
                    dj.append(Df_v[e4, sl])
                accd = mi[0] * dj[0]
                for q in range(1, R_DIM // L):
                    accd = accd + mi[q] * dj[q]
                res = ALPHA * plsc.cumsum(accd)
                for s in range(S):
                    acca = None
                    accc = None
                    for q in range(R_DIM // L):
                        fa = (R_DIM + s + 48 * q) + iota3
                        va = plsc.load_gather(
                            Df_v, [e4 + (fa >> 7), fa & 127])
                        fc = (R_DIM + S * R_DIM + s + 48 * q) + iota3
                        vc = plsc.load_gather(
                            Df_v, [e4 + (fc >> 7), fc & 127])
                        pa = va * mi[q]
                        pc = vc * msum[q]
                        acca = pa if acca is None else acca + pa
                        accc = pc if accc is None else accc + pc
                    res = res + ((BETA * BETA)
                                 * plsc.cumsum(acca) * plsc.cumsum(accc))
                plsc.store_scatter(out_v, [jnp.full((L,), e, jnp.int32)],
                                   res, mask=last)
                return 0

            lax.fori_loop(0, C, elem_body, 0)

            def fin_body(g, _):
                sl = pl.ds(g * L, L)
                sla = pl.ds(ch * C + g * L, L)
                out_v[sl] = out_v[sl] + mb_all[sla] + db_all[sla]
                return 0

            lax.fori_loop(0, C // L, fin_body, 0)
            pltpu.sync_copy(out_v, out_hbm.at[pl.ds(base, C)])

        fire(0, set0)
        pltpu.make_async_copy(mbar_hbm.at[iv_all], mb_all, sem_pre).wait()
        pltpu.make_async_copy(dbar_hbm.at[jv_all], db_all, sem_pre).wait()

        def pair_body(p, _):
            ch0 = 2 * p
            fire(ch0 + 1, set1)
            drain(set0)
            compute(ch0, set0)

            @pl.when(ch0 + 2 < NCH)
            def _():
                fire(ch0 + 2, set0)

            drain(set1)
            compute(ch0 + 1, set1)
            return 0

        lax.fori_loop(0, NCH // 2, pair_body, 0)

    m_tab = _tc_pack_m(jnp.swapaxes(M, 0, 1), 16384)
    d_tab = jnp.reshape(_tc_pack_d(jnp.swapaxes(D_full, 0, 1), 6144),
                        (4 * D_full.shape[0], 128))
    ijk = jnp.asarray(ijk, jnp.int32)
    return sc_kernel(ijk[:, 0], ijk[:, 1], ijk[:, 2], ijk[:, 3], ijk[:, 4],
                     m_bar, d_bar, m_tab, d_tab)


# speedup vs baseline: 1.4131x; 1.0111x over previous
import functools

import jax
import jax.numpy as jnp
from jax import lax
from jax.experimental import pallas as pl
from jax.experimental.pallas import tpu as pltpu
from jax.experimental.pallas import tpu_sc as plsc

ALPHA = 0.001
BETA = 0.001
S = 3
R_DIM = 64
DF_DIM = 448
MP = 128
L = 16


def _tc_pack_d(d_t, bs):
    F, N = d_t.shape
    grid = ((N + bs - 1) // bs,)

    def body(i_ref, o_ref):
        for g in range(4):
            w = min(128, F - g * 128)
            o_ref[:, g, 0:w] = i_ref[g * 128:g * 128 + w, :].T

    return pl.pallas_call(
        body,
        grid=grid,
        in_specs=[pl.BlockSpec((F, bs), lambda b: (0, b))],
        out_specs=pl.BlockSpec((bs, 4, 128), lambda b: (b, 0, 0)),
        out_shape=jax.ShapeDtypeStruct((N, 4, 128), d_t.dtype),
    )(d_t)


def _tc_pack_m(m_t, bs):
    F, N = m_t.shape
    grid = ((N + bs - 1) // bs,)

    def body(i_ref, o_ref):
        o_ref[:, 0:F] = i_ref[...].T

    return pl.pallas_call(
        body,
        grid=grid,
        in_specs=[pl.BlockSpec((F, bs), lambda b: (0, b))],
        out_specs=pl.BlockSpec((bs, MP), lambda b: (b, 0)),
        out_shape=jax.ShapeDtypeStruct((N, MP), m_t.dtype),
    )(m_t)


def kernel(ijk, m_bar, d_bar, M, D_full):
    B = ijk.shape[0]
    info = plsc.get_sparse_core_info()
    NC, NS = info.num_cores, info.num_subcores
    NW = NC * NS
    EPW = B // NW
    C = 32
    NCH = EPW // C

    mesh = plsc.VectorSubcoreMesh(core_axis_name="c", subcore_axis_name="s")

    def buf_set():
        return [
            pltpu.VMEM((C, MP), jnp.float32),
            pltpu.VMEM((C, MP), jnp.float32),
            pltpu.VMEM((C, MP), jnp.float32),
            pltpu.VMEM((C, MP), jnp.float32),
            pltpu.VMEM((4 * C, 128), jnp.float32),
            pltpu.VMEM((C,), jnp.float32),
            pltpu.SemaphoreType.DMA,
        ]

    @functools.partial(
        pl.kernel,
        mesh=mesh,
        out_type=jax.ShapeDtypeStruct((B,), jnp.float32),
        compiler_params=pltpu.CompilerParams(
            use_tc_tiling_on_sc=False, needs_layout_passes=False),
        scratch_types=[
            pltpu.VMEM((EPW,), jnp.int32),
            pltpu.VMEM((EPW,), jnp.int32),
            pltpu.VMEM((EPW,), jnp.int32),
            pltpu.VMEM((EPW,), jnp.int32),
            pltpu.VMEM((EPW,), jnp.int32),
            pltpu.VMEM((4 * EPW,), jnp.int32),
            pltpu.VMEM((EPW,), jnp.float32),
            pltpu.VMEM((EPW,), jnp.float32),
            pltpu.SemaphoreType.DMA,
        ] + buf_set() + buf_set(),
    )
    def sc_kernel(iv_hbm, jv_hbm, k0_hbm, k1_hbm, k2_hbm,
                  mbar_hbm, dbar_hbm, M_hbm, Df_hbm, out_hbm,
                  iv_all, jv_all, k0_all, k1_all, k2_all, jv4_all,
                  mb_all, db_all, sem_pre, *bufs):
        set0, set1 = bufs[:7], bufs[7:]
        wid = lax.axis_index("s") * NC + lax.axis_index("c")
        wbase = pl.multiple_of(wid * EPW, EPW)

        pltpu.sync_copy(iv_hbm.at[pl.ds(wbase, EPW)], iv_all)
        pltpu.sync_copy(jv_hbm.at[pl.ds(wbase, EPW)], jv_all)
        pltpu.sync_copy(k0_hbm.at[pl.ds(wbase, EPW)], k0_all)
        pltpu.sync_copy(k1_hbm.at[pl.ds(wbase, EPW)], k1_all)
        pltpu.sync_copy(k2_hbm.at[pl.ds(wbase, EPW)], k2_all)

        def j4_body(g, _):
            sl = pl.ds(g * L, L)
            elem4 = (g * (4 * L)) + 4 * lax.iota(jnp.int32, L)
            jj4 = 4 * jv_all[sl]
            for t in range(4):
                plsc.store_scatter(jv4_all, [elem4 + t], jj4 + t)
            return 0

        lax.fori_loop(0, EPW // L, j4_body, 0)
        pltpu.async_copy(mbar_hbm.at[iv_all], mb_all, sem_pre)
        pltpu.async_copy(dbar_hbm.at[jv_all], db_all, sem_pre)

        def copies(ch, bufset):
            Mi_v, Mk0_v, Mk1_v, Mk2_v, Df_v, out_v, sem = bufset
            co = ch * C
            return [
                (M_hbm.at[iv_all.at[pl.ds(co, C)]], Mi_v, sem),
                (Df_hbm.at[jv4_all.at[pl.ds(4 * co, 4 * C)]], Df_v, sem),
                (M_hbm.at[k0_all.at[pl.ds(co, C)]], Mk0_v, sem),
                (M_hbm.at[k1_all.at[pl.ds(co, C)]], Mk1_v, sem),
                (M_hbm.at[k2_all.at[pl.ds(co, C)]], Mk2_v, sem),
            ]

        def fire(ch, bufset):
            for src, dst, sem_ in copies(ch, bufset):
                pltpu.async_copy(src, dst, sem_)

        def drain(bufset):
            for src, dst, sem_ in copies(0, bufset):
                pltpu.make_async_copy(src, dst, sem_).wait()

        def compute(ch, bufset):
            Mi_v, Mk0_v, Mk1_v, Mk2_v, Df_v, out_v, sem = bufset
            base = pl.multiple_of(wid * EPW + ch * C, C)
            iota = lax.iota(jnp.int32, L)
            iota3 = 3 * iota
            last = iota == (L - 1)

            def elem_body(e, _):
                e4 = 4 * e
                mi = []
                msum = []
                dj = []
                for q in range(R_DIM // L):
                    sl = pl.ds(q * L, L)
                    mi.append(Mi_v[e, sl])
                    msum.append(Mk0_v[e, sl] + Mk1_v[e, sl] + Mk2_v[e, sl])
                    dj.append(Df_v[e4, sl])
                accd = mi[0] * dj[0]
                for q in range(1, R_DIM // L):
                    accd = accd + mi[q] * dj[q]
                res = ALPHA * plsc.cumsum(accd)
                for s in range(S):
                    acca = None
                    accc = None
                    for q in range(R_DIM // L):
                        fa = (R_DIM + s + 48 * q) + iota3
                        va = plsc.load_gather(
                            Df_v, [e4 + (fa >> 7), fa & 127])
                        fc = (R_DIM + S * R_DIM + s + 48 * q) + iota3
                        vc = plsc.load_gather(
                            Df_v, [e4 + (fc >> 7), fc & 127])
                        pa = va * mi[q]
                        pc = vc * msum[q]
                        acca = pa if acca is None else acca + pa
                        accc = pc if accc is None else accc + pc
                    res = res + ((BETA * BETA)
                                 * plsc.cumsum(acca) * plsc.cumsum(accc))
                plsc.store_scatter(out_v, [jnp.full((L,), e, jnp.int32)],
                                   res, mask=last)
                return 0

            lax.fori_loop(0, C, elem_body, 0)

            def fin_body(g, _):
                sl = pl.ds(g * L, L)
                sla = pl.ds(ch * C + g * L, L)
                out_v[sl] = out_v[sl] + mb_all[sla] + db_all[sla]
                return 0

            lax.fori_loop(0, C // L, fin_body, 0)
            pltpu.sync_copy(out_v, out_hbm.at[pl.ds(base, C)])

        fire(0, set0)
        pltpu.make_async_copy(mbar_hbm.at[iv_all], mb_all, sem_pre).wait()
        pltpu.make_async_copy(dbar_hbm.at[jv_all], db_all, sem_pre).wait()

        def pair_body(p, _):
            ch0 = 2 * p
            fire(ch0 + 1, set1)
            drain(set0)
            compute(ch0, set0)

            @pl.when(ch0 + 2 < NCH)
            def _():
                fire(ch0 + 2, set0)

            drain(set1)
            compute(ch0 + 1, set1)
            return 0

        lax.fori_loop(0, NCH // 2, pair_body, 0)

    m_tab = _tc_pack_m(jnp.swapaxes(M, 0, 1), 16384)
    d_tab = jnp.reshape(_tc_pack_d(jnp.swapaxes(D_full, 0, 1), 7168),
                        (4 * D_full.shape[0], 128))
    ijk = jnp.asarray(ijk, jnp.int32)
    return sc_kernel(ijk[:, 0], ijk[:, 1], ijk[:, 2], ijk[:, 3], ijk[:, 4],
                     m_bar, d_bar, m_tab, d_tab)
